# baseline (speedup 1.0000x reference)
import functools

import jax
import jax.numpy as jnp
from jax import lax
from jax.experimental import pallas as pl
from jax.experimental.pallas import tpu as pltpu
from jax.experimental.pallas import tpu_sc as plsc

NC = 2
NS = 16
NW = NC * NS
L = 16

B, S, D = 4, 2048, 1024
ROWS = B * S
RPW = ROWS // NW
CHUNK = 16
NCHUNK = RPW // CHUNK
CELEMS = CHUNK * D
NXB = 4
NPB = 3

_mesh = plsc.VectorSubcoreMesh(core_axis_name="c", subcore_axis_name="s")


@functools.partial(
    pl.kernel,
    out_type=jax.ShapeDtypeStruct((ROWS * D,), jnp.float32),
    mesh=_mesh,
    scratch_types=(
        [pltpu.VMEM((CELEMS,), jnp.float32)] * (NXB + NPB)
        + [pltpu.SemaphoreType.DMA] * 3
    ),
)
def _pos_add(x_hbm, pe_hbm, out_hbm, *bufs_and_sems):
    xbufs = list(bufs_and_sems[:NXB])
    pbufs = list(bufs_and_sems[NXB:NXB + NPB])
    semx, sempe, semo = bufs_and_sems[NXB + NPB:]
    wid = lax.axis_index("s") * NC + lax.axis_index("c")
    row0 = wid * RPW
    pe_row0 = lax.rem(row0, S)

    def start_in(c):
        base = (row0 + c * CHUNK) * D
        pbase = (pe_row0 + c * CHUNK) * D
        dx = pltpu.async_copy(x_hbm.at[pl.ds(base, CELEMS)],
                              xbufs[c % NXB], semx)
        dp = pltpu.async_copy(pe_hbm.at[pl.ds(pbase, CELEMS)],
                              pbufs[c % NPB], sempe)
        return dx, dp

    in_descs = [start_in(c) for c in range(min(NXB - 1, NCHUNK))]
    out_descs = []
    for c in range(NCHUNK):
        nxt = c + NXB - 1
        if nxt < NCHUNK:
            if nxt >= NXB:
                out_descs[nxt - NXB].wait()
            in_descs.append(start_in(nxt))
        dx, dp = in_descs[c]
        dx.wait()
        dp.wait()
        xbuf = xbufs[c % NXB]
        pbuf = pbufs[c % NPB]

        @plsc.parallel_loop(0, CELEMS, step=L, unroll=8)
        def _add(i):
            plsc.addupdate(xbuf.at[pl.ds(i, L)], pbuf[pl.ds(i, L)])

        base = (row0 + c * CHUNK) * D
        out_descs.append(
            pltpu.async_copy(xbuf, out_hbm.at[pl.ds(base, CELEMS)], semo))
    for d in out_descs[-NXB:]:
        d.wait()


def kernel(x, pos_embedding):
    out = _pos_add(x.reshape(-1), pos_embedding.reshape(-1))
    return out.reshape(x.shape)


# speedup vs baseline: 3.5413x; 3.5413x over previous
import functools

import jax
import jax.numpy as jnp
from jax.experimental import pallas as pl
from jax.experimental.pallas import tpu as pltpu

B, S, D = 4, 2048, 1024
BS = 256


def _add_body(x_ref, pe_ref, o_ref):
    o_ref[...] = x_ref[...] + pe_ref[...][None]


@jax.jit
def _tc_add(x, pos_embedding):
    return pl.pallas_call(
        _add_body,
        grid=(S // BS, B),
        in_specs=[
            pl.BlockSpec((1, BS, D), lambda s, b: (b, s, 0)),
            pl.BlockSpec((BS, D), lambda s, b: (s, 0)),
        ],
        out_specs=pl.BlockSpec((1, BS, D), lambda s, b: (b, s, 0)),
        out_shape=jax.ShapeDtypeStruct((B, S, D), jnp.float32),
        compiler_params=pltpu.CompilerParams(
            dimension_semantics=("arbitrary", "arbitrary")),
    )(x, pos_embedding)


def kernel(x, pos_embedding):
    return _tc_add(x, pos_embedding)
